# Initial kernel scaffold; baseline (speedup 1.0000x reference)
import jax
import jax.numpy as jnp
from jax.experimental import pallas as pl


def kernel(samples, W0, a0, W1, a1, W2, a2, W3, a3, W_last, a_last):
    raise NotImplementedError("write your pallas kernel here")


# speedup vs baseline: 1612.2518x; 1612.2518x over previous
import jax
import jax.numpy as jnp
from jax.experimental import pallas as pl
from jax.experimental.pallas import tpu as pltpu

_NHEAD = 4
_NH = 8
_EN = 2
_ALPHA = 0.2


def _leaky(e):
    return jnp.where(e >= 0, e, _ALPHA * e)


def _elu(r):
    return jnp.where(r > 0, r, jnp.exp(jnp.minimum(r, 0.0)) - 1.0)


def _gat_fused_kernel(x_ref, adj_ref, wall_ref, a_ref, wlast_ref, out_ref):
    f32 = jnp.float32
    n = x_ref.shape[0]
    mask = (adj_ref[...] != 0.0).astype(f32)
    w_all = jnp.dot(x_ref[...], wall_ref[...], preferred_element_type=f32)
    a_cat = a_ref[...]
    ones_col = jnp.ones((n, 1), f32)

    h_parts = []
    for k in range(_NHEAD):
        w_h = w_all[:, k * _NH:(k + 1) * _NH]
        f = jnp.sum(w_h * a_cat[k:k + 1, :], axis=1, keepdims=True)
        g = jax.lax.dot_general(
            a_cat[_NHEAD + k:_NHEAD + k + 1, :], w_h,
            dimension_numbers=(((1,), (1,)), ((), ())),
            preferred_element_type=f32)
        vals = jnp.exp(-_leaky(f + g)) * mask
        aug = jnp.concatenate([w_h, ones_col], axis=1)
        nd = jnp.dot(vals, aug, preferred_element_type=f32)
        h_parts.append(_elu(nd[:, :_NH] / nd[:, _NH:_NH + 1]))

    h = jnp.concatenate(h_parts, axis=1)
    w2 = jnp.dot(h, wlast_ref[...], preferred_element_type=f32)
    f2 = jnp.sum(w2 * a_cat[8:9, :_EN], axis=1, keepdims=True)
    g2 = jax.lax.dot_general(
        a_cat[9:10, :_EN], w2,
        dimension_numbers=(((1,), (1,)), ((), ())),
        preferred_element_type=f32)
    vals2 = jnp.exp(-_leaky(f2 + g2)) * mask
    aug2 = jnp.concatenate([w2, ones_col], axis=1)
    nd2 = jnp.dot(vals2, aug2, preferred_element_type=f32)
    out_ref[...] = _elu(nd2[:, :_EN] / nd2[:, _EN:_EN + 1])


def kernel(samples, W0, a0, W1, a1, W2, a2, W3, a3, W_last, a_last):
    f32 = jnp.float32
    n = samples.shape[2]
    w_all = jnp.concatenate([W0, W1, W2, W3], axis=1)
    heads_a = jnp.concatenate([a0, a1, a2, a3], axis=0)
    a_cat = jnp.zeros((16, _NH), f32)
    a_cat = a_cat.at[0:4, :].set(heads_a[:, :_NH])
    a_cat = a_cat.at[4:8, :].set(heads_a[:, _NH:])
    a_cat = a_cat.at[8, :_EN].set(a_last[0, :_EN])
    a_cat = a_cat.at[9, :_EN].set(a_last[0, _EN:])

    call = pl.pallas_call(
        _gat_fused_kernel,
        out_shape=jax.ShapeDtypeStruct((n, _EN), f32),
        compiler_params=pltpu.CompilerParams(
            vmem_limit_bytes=100 * 1024 * 1024),
    )

    outs = []
    for s in range(samples.shape[0]):
        x = samples[s, 0]
        adj = samples[s, 1]
        outs.append(call(x, adj, w_all, a_cat, W_last))
    return jnp.stack(outs, 0)


# speedup vs baseline: 1668.0264x; 1.0346x over previous
import jax
import jax.numpy as jnp
from jax.experimental import pallas as pl
from jax.experimental.pallas import tpu as pltpu

_NHEAD = 4
_NH = 8
_EN = 2
_ALPHA = 0.2


def _edge_vals(f, g, mask):
    ea, ec = jnp.exp(-f), jnp.exp(-_ALPHA * f)
    eb, ed = jnp.exp(-g), jnp.exp(-_ALPHA * g)
    return jnp.minimum(ea * eb, ec * ed) * mask


def _elu(r):
    return jnp.where(r > 0, r, jnp.exp(jnp.minimum(r, 0.0)) - 1.0)


def _gat_fused_kernel(x_ref, adj_ref, wall_ref, a_ref, wlast_ref, out_ref):
    f32 = jnp.float32
    n = x_ref.shape[0]
    mask = (adj_ref[...] != 0.0).astype(f32)
    w_all = jnp.dot(x_ref[...], wall_ref[...], preferred_element_type=f32)
    a_cat = a_ref[...]
    ones_col = jnp.ones((n, 1), f32)

    h_parts = []
    for k in range(_NHEAD):
        w_h = w_all[:, k * _NH:(k + 1) * _NH]
        f = jnp.sum(w_h * a_cat[k:k + 1, :], axis=1, keepdims=True)
        g = jax.lax.dot_general(
            a_cat[_NHEAD + k:_NHEAD + k + 1, :], w_h,
            dimension_numbers=(((1,), (1,)), ((), ())),
            preferred_element_type=f32)
        vals = _edge_vals(f, g, mask)
        aug = jnp.concatenate([w_h, ones_col], axis=1)
        nd = jnp.dot(vals, aug, preferred_element_type=f32)
        h_parts.append(_elu(nd[:, :_NH] / nd[:, _NH:_NH + 1]))

    h = jnp.concatenate(h_parts, axis=1)
    w2 = jnp.dot(h, wlast_ref[...], preferred_element_type=f32)
    f2 = jnp.sum(w2 * a_cat[8:9, :_EN], axis=1, keepdims=True)
    g2 = jax.lax.dot_general(
        a_cat[9:10, :_EN], w2,
        dimension_numbers=(((1,), (1,)), ((), ())),
        preferred_element_type=f32)
    vals2 = _edge_vals(f2, g2, mask)
    aug2 = jnp.concatenate([w2, ones_col], axis=1)
    nd2 = jnp.dot(vals2, aug2, preferred_element_type=f32)
    out_ref[...] = _elu(nd2[:, :_EN] / nd2[:, _EN:_EN + 1])


def kernel(samples, W0, a0, W1, a1, W2, a2, W3, a3, W_last, a_last):
    f32 = jnp.float32
    n = samples.shape[2]
    w_all = jnp.concatenate([W0, W1, W2, W3], axis=1)
    heads_a = jnp.concatenate([a0, a1, a2, a3], axis=0)
    a_cat = jnp.zeros((16, _NH), f32)
    a_cat = a_cat.at[0:4, :].set(heads_a[:, :_NH])
    a_cat = a_cat.at[4:8, :].set(heads_a[:, _NH:])
    a_cat = a_cat.at[8, :_EN].set(a_last[0, :_EN])
    a_cat = a_cat.at[9, :_EN].set(a_last[0, _EN:])

    call = pl.pallas_call(
        _gat_fused_kernel,
        out_shape=jax.ShapeDtypeStruct((n, _EN), f32),
        compiler_params=pltpu.CompilerParams(
            vmem_limit_bytes=100 * 1024 * 1024),
    )

    outs = []
    for s in range(samples.shape[0]):
        x = samples[s, 0]
        adj = samples[s, 1]
        outs.append(call(x, adj, w_all, a_cat, W_last))
    return jnp.stack(outs, 0)


# speedup vs baseline: 2280.4384x; 1.3671x over previous
import functools

import jax
import jax.numpy as jnp
from jax.experimental import pallas as pl
from jax.experimental.pallas import tpu as pltpu

_NHEAD = 4
_NH = 8
_EN = 2
_ALPHA = 0.2


def _edge_vals(f, g, mask):
    ea, ec = jnp.exp(-f), jnp.exp(-_ALPHA * f)
    eb, ed = jnp.exp(-g), jnp.exp(-_ALPHA * g)
    return jnp.minimum(ea * eb, ec * ed) * mask


def _elu(r):
    return jnp.where(r > 0, r, jnp.exp(jnp.minimum(r, 0.0)) - 1.0)


def _gat_fused_kernel(samples_ref, wall_ref, a_ref, wlast_ref, out_ref, *, s):
    f32 = jnp.float32
    n = samples_ref.shape[-1]
    mask = (samples_ref[s, 1] != 0.0).astype(f32)
    w_all = jnp.dot(samples_ref[s, 0], wall_ref[...], preferred_element_type=f32)
    a_cat = a_ref[...]
    ones_col = jnp.ones((n, 1), f32)

    h_parts = []
    for k in range(_NHEAD):
        w_h = w_all[:, k * _NH:(k + 1) * _NH]
        f = jnp.sum(w_h * a_cat[k:k + 1, :], axis=1, keepdims=True)
        g = jax.lax.dot_general(
            a_cat[_NHEAD + k:_NHEAD + k + 1, :], w_h,
            dimension_numbers=(((1,), (1,)), ((), ())),
            preferred_element_type=f32)
        vals = _edge_vals(f, g, mask)
        aug = jnp.concatenate([w_h, ones_col], axis=1)
        nd = jnp.dot(vals, aug, preferred_element_type=f32)
        h_parts.append(_elu(nd[:, :_NH] / nd[:, _NH:_NH + 1]))

    h = jnp.concatenate(h_parts, axis=1)
    w2 = jnp.dot(h, wlast_ref[...], preferred_element_type=f32)
    f2 = jnp.sum(w2 * a_cat[8:9, :_EN], axis=1, keepdims=True)
    g2 = jax.lax.dot_general(
        a_cat[9:10, :_EN], w2,
        dimension_numbers=(((1,), (1,)), ((), ())),
        preferred_element_type=f32)
    vals2 = _edge_vals(f2, g2, mask)
    aug2 = jnp.concatenate([w2, ones_col], axis=1)
    nd2 = jnp.dot(vals2, aug2, preferred_element_type=f32)
    out_ref[...] = _elu(nd2[:, :_EN] / nd2[:, _EN:_EN + 1])


def kernel(samples, W0, a0, W1, a1, W2, a2, W3, a3, W_last, a_last):
    f32 = jnp.float32
    n = samples.shape[2]
    w_all = jnp.concatenate([W0, W1, W2, W3], axis=1)
    heads_a = jnp.concatenate([a0, a1, a2, a3], axis=0)
    a_cat = jnp.zeros((16, _NH), f32)
    a_cat = a_cat.at[0:4, :].set(heads_a[:, :_NH])
    a_cat = a_cat.at[4:8, :].set(heads_a[:, _NH:])
    a_cat = a_cat.at[8, :_EN].set(a_last[0, :_EN])
    a_cat = a_cat.at[9, :_EN].set(a_last[0, _EN:])

    outs = []
    for s in range(samples.shape[0]):
        call = pl.pallas_call(
            functools.partial(_gat_fused_kernel, s=s),
            out_shape=jax.ShapeDtypeStruct((n, _EN), f32),
            compiler_params=pltpu.CompilerParams(
                vmem_limit_bytes=100 * 1024 * 1024),
        )
        outs.append(call(samples, w_all, a_cat, W_last))
    return jnp.stack(outs, 0)
